# Initial kernel scaffold; baseline (speedup 1.0000x reference)
import jax
import jax.numpy as jnp
from jax.experimental import pallas as pl


def kernel(localizations, classifications, localizations_default):
    raise NotImplementedError("write your pallas kernel here")


# speedup vs baseline: 3.2443x; 3.2443x over previous
import functools

import jax
import jax.numpy as jnp
from jax import lax
from jax.experimental import pallas as pl
from jax.experimental.pallas import tpu as pltpu
from jax.experimental.pallas import tpu_sc as plsc

NUM_CLASSES = 21
OVERLAP = 0.45
TOP_K = 200
CLS_THRESHOLD = 0.01

_B, _N, _C = 8, 20000, NUM_CLASSES
_NBLK = 2000


def _stage_a_body(cls_ref, loc_ref, pri_ref, sc_ref, dec_ref):
    x = cls_ref[0]
    m = jnp.max(x, axis=-1, keepdims=True)
    e = jnp.exp(x - m)
    s = jnp.sum(e, axis=-1, keepdims=True)
    p = e / s
    sc_ref[0] = p[:, 1:]

    l = loc_ref[0]
    pr = pri_ref[...]
    center = pr[:, 0:1] + l[:, 0:1] * 0.1 * pr[:, 1:2]
    width = pr[:, 1:2] * jnp.exp(l[:, 1:2] * 0.2)
    half = width / 2.0
    dec_ref[0] = jnp.concatenate([center - half, center + half], axis=1)


def _stage_a(cls, loc, pri):
    grid = (_B, _N // _NBLK)
    return pl.pallas_call(
        _stage_a_body,
        grid=grid,
        in_specs=[
            pl.BlockSpec((1, _NBLK, _C), lambda b, n: (b, n, 0)),
            pl.BlockSpec((1, _NBLK, 2), lambda b, n: (b, n, 0)),
            pl.BlockSpec((_NBLK, 2), lambda b, n: (n, 0)),
        ],
        out_specs=[
            pl.BlockSpec((1, _NBLK, _C - 1), lambda b, n: (b, n, 0)),
            pl.BlockSpec((1, _NBLK, 2), lambda b, n: (b, n, 0)),
        ],
        out_shape=[
            jax.ShapeDtypeStruct((_B, _N, _C - 1), jnp.float32),
            jax.ShapeDtypeStruct((_B, _N, 2), jnp.float32),
        ],
    )(cls, loc, pri)


def _xla_tail(scores, decoded):
    B, N, Cm1 = scores.shape
    K = TOP_K
    scores_t = jnp.transpose(scores, (0, 2, 1))
    top_scores, top_idx = jax.lax.top_k(scores_t, K)
    decoded_b = jnp.broadcast_to(decoded[:, None, :, :], (B, Cm1, N, 2))
    boxes = jnp.take_along_axis(decoded_b, top_idx[..., None], axis=2)
    valid = top_scores > CLS_THRESHOLD
    s = boxes[..., 0]
    e = boxes[..., 1]
    lengths = jnp.clip(e - s, 0.0)
    inter = jnp.clip(
        jnp.minimum(e[..., :, None], e[..., None, :])
        - jnp.maximum(s[..., :, None], s[..., None, :]), 0.0)
    union = lengths[..., :, None] + lengths[..., None, :] - inter
    iou = inter / (union + 1e-9)
    keep = valid
    idxr = jnp.arange(K)
    for i in range(K):
        cur = keep[..., i]
        supp = (iou[..., i, :] > OVERLAP) & (idxr > i)
        keep = keep & ~(cur[..., None] & supp)
    out = jnp.concatenate([boxes, top_scores[..., None]], axis=-1)
    return out * keep[..., None].astype(out.dtype)


_CM1 = _C - 1
_K = TOP_K
_KPAD = 208
_CAP = 512
_CAPP = _CAP + 16
_ROW = 640
_NV = _N // 16
_CHUNK = 10
_BIG_I = 2**30


def _count_gt(sc_ref, thr):
    def outer(i, acc):
        base = i * (16 * _CHUNK)
        for u in range(_CHUNK):
            v = sc_ref[pl.ds(base + u * 16, 16)]
            acc = acc + jnp.where(v > thr, jnp.int32(1), jnp.int32(0))
        return acc
    acc = lax.fori_loop(0, _NV // _CHUNK, outer,
                        jnp.zeros((16,), jnp.int32))
    return jnp.sum(acc)


def _stage_b_body(sct_ref, dstart_ref, dend_ref, out_ref,
                  sc_v, ds_v, de_v, cs_v, ci_v, ss_v, si_v,
                  bs_v, be_v, kp_v, ov_v):
    nc = 2
    wid = lax.axis_index("s") * nc + lax.axis_index("c")
    b = wid // 4
    cg = wid % 4

    lane = lax.iota(jnp.int32, 16)
    mask0 = lane == 0

    pltpu.sync_copy(dstart_ref.at[pl.ds(b * _N, _N)], ds_v)
    pltpu.sync_copy(dend_ref.at[pl.ds(b * _N, _N)], de_v)

    for k in range(_CM1 // 4):
        c = cg * 5 + k
        pltpu.sync_copy(sct_ref.at[pl.ds((b * _CM1 + c) * _N, _N)], sc_v)

        def bs_body(_, st):
            lo, hi, cnt_lo = st
            done = cnt_lo <= 448
            mid = (lo + hi) * 0.5
            cnt = _count_gt(sc_v, mid)
            ge = cnt >= 200
            lo2 = jnp.where(jnp.logical_or(done, jnp.logical_not(ge)), lo, mid)
            hi2 = jnp.where(jnp.logical_or(done, ge), hi, mid)
            cnt2 = jnp.where(jnp.logical_or(done, jnp.logical_not(ge)),
                             cnt_lo, cnt)
            return (lo2, hi2, cnt2)

        lo, _, _ = lax.fori_loop(
            0, 28, bs_body,
            (jnp.float32(0.0), jnp.float32(1.0), jnp.int32(_N)))

        for j in range(_CAPP // 16):
            cs_v[pl.ds(j * 16, 16)] = jnp.full((16,), -1.0, jnp.float32)
            ci_v[pl.ds(j * 16, 16)] = jnp.full((16,), _BIG_I, jnp.int32)

        def comp_outer(i, off):
            base = i * (16 * _CHUNK)
            for u in range(_CHUNK):
                v = sc_v[pl.ds(base + u * 16, 16)]
                m = v > lo
                idxv = lane + (base + u * 16)
                cnt = jnp.sum(jnp.where(m, jnp.int32(1), jnp.int32(0)))
                @pl.when(off <= _CAP - 16)
                def _():
                    plsc.store_compressed(cs_v.at[pl.ds(off, 16)], v, mask=m)
                    plsc.store_compressed(ci_v.at[pl.ds(off, 16)], idxv, mask=m)
                off = jnp.minimum(off + cnt, jnp.int32(_CAP))
            return off
        lax.fori_loop(0, _NV // _CHUNK, comp_outer, jnp.int32(0))

        for j in range(_KPAD // 16):
            ss_v[pl.ds(j * 16, 16)] = jnp.full((16,), -1.0, jnp.float32)
            si_v[pl.ds(j * 16, 16)] = jnp.zeros((16,), jnp.int32)

        def pick(i, _):
            def mx(j, macc):
                return jnp.maximum(macc, cs_v[pl.ds(j * 16, 16)])
            macc = lax.fori_loop(0, _CAPP // 16, mx,
                                 jnp.full((16,), -2.0, jnp.float32))
            mval = jnp.max(macc)

            def mi(j, iacc):
                v = cs_v[pl.ds(j * 16, 16)]
                ix = ci_v[pl.ds(j * 16, 16)]
                return jnp.minimum(iacc, jnp.where(v == mval, ix, jnp.int32(_BIG_I)))
            imin = jnp.min(lax.fori_loop(0, _CAPP // 16, mi,
                                         jnp.full((16,), _BIG_I, jnp.int32)))

            plsc.store_scatter(ss_v, [jnp.full((16,), i, jnp.int32)],
                               jnp.full((16,), mval, jnp.float32), mask=mask0)
            imin_c = jnp.minimum(imin, jnp.int32(_N - 1))
            plsc.store_scatter(si_v, [jnp.full((16,), i, jnp.int32)],
                               jnp.full((16,), imin_c, jnp.int32), mask=mask0)

            def clr(j, _):
                v = cs_v[pl.ds(j * 16, 16)]
                ix = ci_v[pl.ds(j * 16, 16)]
                cs_v[pl.ds(j * 16, 16)] = jnp.where(ix == imin, -2.0, v)
                return 0
            lax.fori_loop(0, _CAPP // 16, clr, 0)
            return 0
        lax.fori_loop(0, _K, pick, 0)

        for j in range(_KPAD // 16):
            idxv = si_v[pl.ds(j * 16, 16)]
            bs_v[pl.ds(j * 16, 16)] = plsc.load_gather(ds_v, [idxv])
            be_v[pl.ds(j * 16, 16)] = plsc.load_gather(de_v, [idxv])
            sv = ss_v[pl.ds(j * 16, 16)]
            kp_v[pl.ds(j * 16, 16)] = jnp.where(
                sv > CLS_THRESHOLD, 1.0, 0.0).astype(jnp.float32)

        def nms(i, _):
            base = (i // 16) * 16
            li = i - base
            mlane = lane == li
            bsv = bs_v[pl.ds(base, 16)]
            bev = be_v[pl.ds(base, 16)]
            kpv = kp_v[pl.ds(base, 16)]
            s_i = jnp.sum(jnp.where(mlane, bsv, 0.0))
            e_i = jnp.sum(jnp.where(mlane, bev, 0.0))
            cur = jnp.sum(jnp.where(mlane, kpv, 0.0))
            len_i = jnp.maximum(e_i - s_i, 0.0)
            for j in range(_KPAD // 16):
                sv = bs_v[pl.ds(j * 16, 16)]
                ev = be_v[pl.ds(j * 16, 16)]
                kv = kp_v[pl.ds(j * 16, 16)]
                inter = jnp.maximum(
                    jnp.minimum(ev, e_i) - jnp.maximum(sv, s_i), 0.0)
                union = jnp.maximum(ev - sv, 0.0) + len_i - inter
                iou = inter / (union + 1e-9)
                jvec = lane + (j * 16)
                supp = jnp.logical_and(iou > OVERLAP, jvec > i)
                supp = jnp.logical_and(supp, cur > 0.5)
                kp_v[pl.ds(j * 16, 16)] = jnp.where(supp, 0.0, kv)
            return 0
        lax.fori_loop(0, _K, nms, 0)

        for j in range(_ROW // 16):
            ov_v[pl.ds(j * 16, 16)] = jnp.zeros((16,), jnp.float32)
        for j in range(_KPAD // 16):
            sl = pl.ds(j * 16, 16)
            kv = kp_v[sl]
            pos3 = (lane + j * 16) * 3
            mvalid = pos3 < (_K * 3)
            plsc.store_scatter(ov_v, [jnp.minimum(pos3, _K * 3 - 1)],
                               bs_v[sl] * kv, mask=mvalid)
            plsc.store_scatter(ov_v, [jnp.minimum(pos3 + 1, _K * 3 - 1)],
                               be_v[sl] * kv, mask=mvalid)
            plsc.store_scatter(ov_v, [jnp.minimum(pos3 + 2, _K * 3 - 1)],
                               ss_v[sl] * kv, mask=mvalid)
        pltpu.sync_copy(ov_v, out_ref.at[pl.ds((b * _CM1 + c) * _ROW, _ROW)])


def _stage_b(scores_t, dec_start, dec_end):
    mesh = plsc.VectorSubcoreMesh(core_axis_name="c", subcore_axis_name="s",
                                  num_cores=2, num_subcores=16)
    f = pl.kernel(
        _stage_b_body,
        out_type=jax.ShapeDtypeStruct((_B * _CM1 * _ROW,), jnp.float32),
        mesh=mesh,
        scratch_types=[
            pltpu.VMEM((_N,), jnp.float32),
            pltpu.VMEM((_N,), jnp.float32),
            pltpu.VMEM((_N,), jnp.float32),
            pltpu.VMEM((_CAPP,), jnp.float32),
            pltpu.VMEM((_CAPP,), jnp.int32),
            pltpu.VMEM((_KPAD,), jnp.float32),
            pltpu.VMEM((_KPAD,), jnp.int32),
            pltpu.VMEM((_KPAD,), jnp.float32),
            pltpu.VMEM((_KPAD,), jnp.float32),
            pltpu.VMEM((_KPAD,), jnp.float32),
            pltpu.VMEM((_ROW,), jnp.float32),
        ],
        compiler_params=pltpu.CompilerParams(needs_layout_passes=False),
    )
    return f(scores_t.reshape(-1), dec_start.reshape(-1), dec_end.reshape(-1))


def kernel(localizations, classifications, localizations_default):
    scores, decoded = _stage_a(classifications, localizations,
                               localizations_default)
    scores_t = jnp.transpose(scores, (0, 2, 1))
    out = _stage_b(scores_t, decoded[:, :, 0], decoded[:, :, 1])
    out = out.reshape(_B, _CM1, _ROW)[:, :, :_K * 3]
    return out.reshape(_B, _CM1, _K, 3)


# speedup vs baseline: 4.0090x; 1.2357x over previous
import functools

import jax
import jax.numpy as jnp
from jax import lax
from jax.experimental import pallas as pl
from jax.experimental.pallas import tpu as pltpu
from jax.experimental.pallas import tpu_sc as plsc

NUM_CLASSES = 21
OVERLAP = 0.45
TOP_K = 200
CLS_THRESHOLD = 0.01

_B, _N, _C = 8, 20000, NUM_CLASSES
_NBLK = 2000


def _stage_a_body(cls_ref, loc_ref, pri_ref, sc_ref, dec_ref):
    x = cls_ref[0]
    m = jnp.max(x, axis=-1, keepdims=True)
    e = jnp.exp(x - m)
    s = jnp.sum(e, axis=-1, keepdims=True)
    p = e / s
    sc_ref[0] = p[:, 1:]

    l = loc_ref[0]
    pr = pri_ref[...]
    center = pr[:, 0:1] + l[:, 0:1] * 0.1 * pr[:, 1:2]
    width = pr[:, 1:2] * jnp.exp(l[:, 1:2] * 0.2)
    half = width / 2.0
    dec_ref[0] = jnp.concatenate([center - half, center + half], axis=1)


def _stage_a(cls, loc, pri):
    grid = (_B, _N // _NBLK)
    return pl.pallas_call(
        _stage_a_body,
        grid=grid,
        in_specs=[
            pl.BlockSpec((1, _NBLK, _C), lambda b, n: (b, n, 0)),
            pl.BlockSpec((1, _NBLK, 2), lambda b, n: (b, n, 0)),
            pl.BlockSpec((_NBLK, 2), lambda b, n: (n, 0)),
        ],
        out_specs=[
            pl.BlockSpec((1, _NBLK, _C - 1), lambda b, n: (b, n, 0)),
            pl.BlockSpec((1, _NBLK, 2), lambda b, n: (b, n, 0)),
        ],
        out_shape=[
            jax.ShapeDtypeStruct((_B, _N, _C - 1), jnp.float32),
            jax.ShapeDtypeStruct((_B, _N, 2), jnp.float32),
        ],
    )(cls, loc, pri)


def _xla_tail(scores, decoded):
    B, N, Cm1 = scores.shape
    K = TOP_K
    scores_t = jnp.transpose(scores, (0, 2, 1))
    top_scores, top_idx = jax.lax.top_k(scores_t, K)
    decoded_b = jnp.broadcast_to(decoded[:, None, :, :], (B, Cm1, N, 2))
    boxes = jnp.take_along_axis(decoded_b, top_idx[..., None], axis=2)
    valid = top_scores > CLS_THRESHOLD
    s = boxes[..., 0]
    e = boxes[..., 1]
    lengths = jnp.clip(e - s, 0.0)
    inter = jnp.clip(
        jnp.minimum(e[..., :, None], e[..., None, :])
        - jnp.maximum(s[..., :, None], s[..., None, :]), 0.0)
    union = lengths[..., :, None] + lengths[..., None, :] - inter
    iou = inter / (union + 1e-9)
    keep = valid
    idxr = jnp.arange(K)
    for i in range(K):
        cur = keep[..., i]
        supp = (iou[..., i, :] > OVERLAP) & (idxr > i)
        keep = keep & ~(cur[..., None] & supp)
    out = jnp.concatenate([boxes, top_scores[..., None]], axis=-1)
    return out * keep[..., None].astype(out.dtype)


_CM1 = _C - 1
_K = TOP_K
_KPAD = 208
_CAP = 512
_CAPP = _CAP + 16
_ROW = 640
_NV = _N // 16
_CHUNK = 10
_BIG_I = 2**30


def _count_gt(sc_ref, thr):
    def outer(i, acc):
        base = i * (16 * _CHUNK)
        for u in range(_CHUNK):
            v = sc_ref[pl.ds(base + u * 16, 16)]
            acc = acc + jnp.where(v > thr, jnp.int32(1), jnp.int32(0))
        return acc
    acc = lax.fori_loop(0, _NV // _CHUNK, outer,
                        jnp.zeros((16,), jnp.int32))
    return jnp.sum(acc)


def _stage_b_body(sct_ref, dstart_ref, dend_ref, out_ref,
                  sc_v, ds_v, de_v, cs_v, ci_v, ss_v, si_v,
                  bs_v, be_v, kp_v, ov_v, tb_v):
    nc = 2
    wid = lax.axis_index("s") * nc + lax.axis_index("c")
    b = wid // 4
    cg = wid % 4

    lane = lax.iota(jnp.int32, 16)
    mask0 = lane == 0

    pltpu.sync_copy(dstart_ref.at[pl.ds(b * _N, _N)], ds_v)
    pltpu.sync_copy(dend_ref.at[pl.ds(b * _N, _N)], de_v)

    for k in range(_CM1 // 4):
        c = cg * 5 + k
        pltpu.sync_copy(sct_ref.at[pl.ds((b * _CM1 + c) * _N, _N)], sc_v)

        def bs_body(_, st):
            lo, hi, cnt_lo = st
            done = cnt_lo <= 448
            mid = (lo + hi) * 0.5
            @pl.when(jnp.logical_not(done))
            def _():
                tb_v[pl.ds(0, 16)] = jnp.full(
                    (16,), _count_gt(sc_v, mid), jnp.int32)
            cnt = tb_v[pl.ds(0, 16)][0]
            ge = cnt >= 200
            lo2 = jnp.where(jnp.logical_or(done, jnp.logical_not(ge)), lo, mid)
            hi2 = jnp.where(jnp.logical_or(done, ge), hi, mid)
            cnt2 = jnp.where(jnp.logical_or(done, jnp.logical_not(ge)),
                             cnt_lo, cnt)
            return (lo2, hi2, cnt2)

        lo, _, _ = lax.fori_loop(
            0, 26, bs_body,
            (jnp.float32(0.0), jnp.float32(1.0), jnp.int32(_N)))

        for j in range(_CAPP // 16):
            cs_v[pl.ds(j * 16, 16)] = jnp.full((16,), -1.0, jnp.float32)
            ci_v[pl.ds(j * 16, 16)] = jnp.full((16,), _BIG_I, jnp.int32)

        def comp_outer(i, off):
            base = i * (16 * _CHUNK)
            for u in range(_CHUNK):
                v = sc_v[pl.ds(base + u * 16, 16)]
                m = v > lo
                idxv = lane + (base + u * 16)
                cnt = jnp.sum(jnp.where(m, jnp.int32(1), jnp.int32(0)))
                @pl.when(off <= _CAP - 16)
                def _():
                    plsc.store_compressed(cs_v.at[pl.ds(off, 16)], v, mask=m)
                    plsc.store_compressed(ci_v.at[pl.ds(off, 16)], idxv, mask=m)
                off = jnp.minimum(off + cnt, jnp.int32(_CAP))
            return off
        ncand = lax.fori_loop(0, _NV // _CHUNK, comp_outer, jnp.int32(0))
        nv = (ncand + 15) // 16

        for j in range(_KPAD // 16):
            ss_v[pl.ds(j * 16, 16)] = jnp.full((16,), -1.0, jnp.float32)
            si_v[pl.ds(j * 16, 16)] = jnp.zeros((16,), jnp.int32)

        def pick(i, _):
            def mx(j, macc):
                return jnp.maximum(macc, cs_v[pl.ds(j * 16, 16)])
            macc = lax.fori_loop(0, nv, mx,
                                 jnp.full((16,), -2.0, jnp.float32))
            mval = jnp.max(macc)

            def mi(j, iacc):
                v = cs_v[pl.ds(j * 16, 16)]
                ix = ci_v[pl.ds(j * 16, 16)]
                return jnp.minimum(iacc, jnp.where(v == mval, ix, jnp.int32(_BIG_I)))
            imin = jnp.min(lax.fori_loop(0, nv, mi,
                                         jnp.full((16,), _BIG_I, jnp.int32)))

            plsc.store_scatter(ss_v, [jnp.full((16,), i, jnp.int32)],
                               jnp.full((16,), mval, jnp.float32), mask=mask0)
            imin_c = jnp.minimum(imin, jnp.int32(_N - 1))
            plsc.store_scatter(si_v, [jnp.full((16,), i, jnp.int32)],
                               jnp.full((16,), imin_c, jnp.int32), mask=mask0)

            def clr(j, _):
                v = cs_v[pl.ds(j * 16, 16)]
                ix = ci_v[pl.ds(j * 16, 16)]
                cs_v[pl.ds(j * 16, 16)] = jnp.where(ix == imin, -2.0, v)
                return 0
            lax.fori_loop(0, nv, clr, 0)
            return 0
        lax.fori_loop(0, _K, pick, 0)

        for j in range(_KPAD // 16):
            idxv = si_v[pl.ds(j * 16, 16)]
            bs_v[pl.ds(j * 16, 16)] = plsc.load_gather(ds_v, [idxv])
            be_v[pl.ds(j * 16, 16)] = plsc.load_gather(de_v, [idxv])
            sv = ss_v[pl.ds(j * 16, 16)]
            kp_v[pl.ds(j * 16, 16)] = jnp.where(
                sv > CLS_THRESHOLD, 1.0, 0.0).astype(jnp.float32)

        def nms(i, _):
            base = (i // 16) * 16
            li = i - base
            mlane = lane == li
            bsv = bs_v[pl.ds(base, 16)]
            bev = be_v[pl.ds(base, 16)]
            kpv = kp_v[pl.ds(base, 16)]
            s_i = jnp.sum(jnp.where(mlane, bsv, 0.0))
            e_i = jnp.sum(jnp.where(mlane, bev, 0.0))
            cur = jnp.sum(jnp.where(mlane, kpv, 0.0))
            len_i = jnp.maximum(e_i - s_i, 0.0)
            @pl.when(cur > 0.5)
            def _():
                for j in range(_KPAD // 16):
                    sv = bs_v[pl.ds(j * 16, 16)]
                    ev = be_v[pl.ds(j * 16, 16)]
                    kv = kp_v[pl.ds(j * 16, 16)]
                    inter = jnp.maximum(
                        jnp.minimum(ev, e_i) - jnp.maximum(sv, s_i), 0.0)
                    union = jnp.maximum(ev - sv, 0.0) + len_i - inter
                    iou = inter / (union + 1e-9)
                    jvec = lane + (j * 16)
                    supp = jnp.logical_and(iou > OVERLAP, jvec > i)
                    kp_v[pl.ds(j * 16, 16)] = jnp.where(supp, 0.0, kv)
            return 0
        lax.fori_loop(0, _K, nms, 0)

        for j in range(_ROW // 16):
            ov_v[pl.ds(j * 16, 16)] = jnp.zeros((16,), jnp.float32)
        for j in range(_KPAD // 16):
            sl = pl.ds(j * 16, 16)
            kv = kp_v[sl]
            pos3 = (lane + j * 16) * 3
            mvalid = pos3 < (_K * 3)
            plsc.store_scatter(ov_v, [jnp.minimum(pos3, _K * 3 - 1)],
                               bs_v[sl] * kv, mask=mvalid)
            plsc.store_scatter(ov_v, [jnp.minimum(pos3 + 1, _K * 3 - 1)],
                               be_v[sl] * kv, mask=mvalid)
            plsc.store_scatter(ov_v, [jnp.minimum(pos3 + 2, _K * 3 - 1)],
                               ss_v[sl] * kv, mask=mvalid)
        pltpu.sync_copy(ov_v, out_ref.at[pl.ds((b * _CM1 + c) * _ROW, _ROW)])


def _stage_b(scores_t, dec_start, dec_end):
    mesh = plsc.VectorSubcoreMesh(core_axis_name="c", subcore_axis_name="s",
                                  num_cores=2, num_subcores=16)
    f = pl.kernel(
        _stage_b_body,
        out_type=jax.ShapeDtypeStruct((_B * _CM1 * _ROW,), jnp.float32),
        mesh=mesh,
        scratch_types=[
            pltpu.VMEM((_N,), jnp.float32),
            pltpu.VMEM((_N,), jnp.float32),
            pltpu.VMEM((_N,), jnp.float32),
            pltpu.VMEM((_CAPP,), jnp.float32),
            pltpu.VMEM((_CAPP,), jnp.int32),
            pltpu.VMEM((_KPAD,), jnp.float32),
            pltpu.VMEM((_KPAD,), jnp.int32),
            pltpu.VMEM((_KPAD,), jnp.float32),
            pltpu.VMEM((_KPAD,), jnp.float32),
            pltpu.VMEM((_KPAD,), jnp.float32),
            pltpu.VMEM((_ROW,), jnp.float32),
            pltpu.VMEM((16,), jnp.int32),
        ],
        compiler_params=pltpu.CompilerParams(needs_layout_passes=False),
    )
    return f(scores_t.reshape(-1), dec_start.reshape(-1), dec_end.reshape(-1))


def kernel(localizations, classifications, localizations_default):
    scores, decoded = _stage_a(classifications, localizations,
                               localizations_default)
    scores_t = jnp.transpose(scores, (0, 2, 1))
    out = _stage_b(scores_t, decoded[:, :, 0], decoded[:, :, 1])
    out = out.reshape(_B, _CM1, _ROW)[:, :, :_K * 3]
    return out.reshape(_B, _CM1, _K, 3)


# speedup vs baseline: 4.2480x; 1.0596x over previous
import functools

import jax
import jax.numpy as jnp
from jax import lax
from jax.experimental import pallas as pl
from jax.experimental.pallas import tpu as pltpu
from jax.experimental.pallas import tpu_sc as plsc

NUM_CLASSES = 21
OVERLAP = 0.45
TOP_K = 200
CLS_THRESHOLD = 0.01

_B, _N, _C = 8, 20000, NUM_CLASSES
_NBLK = 5120


def _stage_a_body(cls_ref, loc_ref, pri_ref, sc_ref, dec_ref):
    x = cls_ref[0]
    m = jnp.max(x, axis=-1, keepdims=True)
    e = jnp.exp(x - m)
    s = jnp.sum(e, axis=-1, keepdims=True)
    p = e / s
    sc_ref[0] = jnp.transpose(p[:, 1:], (1, 0))

    l = loc_ref[0]
    pr = pri_ref[...]
    center = pr[:, 0:1] + l[:, 0:1] * 0.1 * pr[:, 1:2]
    width = pr[:, 1:2] * jnp.exp(l[:, 1:2] * 0.2)
    half = width / 2.0
    dec_ref[0] = jnp.concatenate([center - half, center + half], axis=1)


def _stage_a(cls, loc, pri):
    grid = (_B, (_N + _NBLK - 1) // _NBLK)
    return pl.pallas_call(
        _stage_a_body,
        grid=grid,
        in_specs=[
            pl.BlockSpec((1, _NBLK, _C), lambda b, n: (b, n, 0)),
            pl.BlockSpec((1, _NBLK, 2), lambda b, n: (b, n, 0)),
            pl.BlockSpec((_NBLK, 2), lambda b, n: (n, 0)),
        ],
        out_specs=[
            pl.BlockSpec((1, _C - 1, _NBLK), lambda b, n: (b, 0, n)),
            pl.BlockSpec((1, _NBLK, 2), lambda b, n: (b, n, 0)),
        ],
        out_shape=[
            jax.ShapeDtypeStruct((_B, _C - 1, _N), jnp.float32),
            jax.ShapeDtypeStruct((_B, _N, 2), jnp.float32),
        ],
    )(cls, loc, pri)


def _xla_tail(scores, decoded):
    B, N, Cm1 = scores.shape
    K = TOP_K
    scores_t = jnp.transpose(scores, (0, 2, 1))
    top_scores, top_idx = jax.lax.top_k(scores_t, K)
    decoded_b = jnp.broadcast_to(decoded[:, None, :, :], (B, Cm1, N, 2))
    boxes = jnp.take_along_axis(decoded_b, top_idx[..., None], axis=2)
    valid = top_scores > CLS_THRESHOLD
    s = boxes[..., 0]
    e = boxes[..., 1]
    lengths = jnp.clip(e - s, 0.0)
    inter = jnp.clip(
        jnp.minimum(e[..., :, None], e[..., None, :])
        - jnp.maximum(s[..., :, None], s[..., None, :]), 0.0)
    union = lengths[..., :, None] + lengths[..., None, :] - inter
    iou = inter / (union + 1e-9)
    keep = valid
    idxr = jnp.arange(K)
    for i in range(K):
        cur = keep[..., i]
        supp = (iou[..., i, :] > OVERLAP) & (idxr > i)
        keep = keep & ~(cur[..., None] & supp)
    out = jnp.concatenate([boxes, top_scores[..., None]], axis=-1)
    return out * keep[..., None].astype(out.dtype)


_CM1 = _C - 1
_K = TOP_K
_KPAD = 208
_CAP = 512
_CAPP = _CAP + 16
_ROW = 640
_NV = _N // 16
_CHUNK = 10
_BIG_I = 2**30


def _count_gt(sc_ref, thr):
    def outer(i, acc):
        base = i * (16 * _CHUNK)
        for u in range(_CHUNK):
            v = sc_ref[pl.ds(base + u * 16, 16)]
            acc = acc + jnp.where(v > thr, jnp.int32(1), jnp.int32(0))
        return acc
    acc = lax.fori_loop(0, _NV // _CHUNK, outer,
                        jnp.zeros((16,), jnp.int32))
    return jnp.sum(acc)


def _stage_b_body(sct_ref, dec_ref, out_ref,
                  sc_v, dv_v, cs_v, ci_v, ss_v, si_v,
                  bs_v, be_v, kp_v, ov_v, tb_v):
    nc = 2
    wid = lax.axis_index("s") * nc + lax.axis_index("c")
    b = wid // 4
    cg = wid % 4

    lane = lax.iota(jnp.int32, 16)
    mask0 = lane == 0

    pltpu.sync_copy(dec_ref.at[pl.ds(b * 2 * _N, 2 * _N)], dv_v)

    for k in range(_CM1 // 4):
        c = cg * 5 + k
        pltpu.sync_copy(sct_ref.at[pl.ds((b * _CM1 + c) * _N, _N)], sc_v)

        def bs_body(_, st):
            lo, hi, cnt_lo = st
            done = cnt_lo <= 448
            mid = (lo + hi) * 0.5
            @pl.when(jnp.logical_not(done))
            def _():
                tb_v[pl.ds(0, 16)] = jnp.full(
                    (16,), _count_gt(sc_v, mid), jnp.int32)
            cnt = tb_v[pl.ds(0, 16)][0]
            ge = cnt >= 200
            lo2 = jnp.where(jnp.logical_or(done, jnp.logical_not(ge)), lo, mid)
            hi2 = jnp.where(jnp.logical_or(done, ge), hi, mid)
            cnt2 = jnp.where(jnp.logical_or(done, jnp.logical_not(ge)),
                             cnt_lo, cnt)
            return (lo2, hi2, cnt2)

        lo, _, _ = lax.fori_loop(
            0, 26, bs_body,
            (jnp.float32(0.0), jnp.float32(1.0), jnp.int32(_N)))

        for j in range(_CAPP // 16):
            cs_v[pl.ds(j * 16, 16)] = jnp.full((16,), -1.0, jnp.float32)
            ci_v[pl.ds(j * 16, 16)] = jnp.full((16,), _BIG_I, jnp.int32)

        def comp_outer(i, off):
            base = i * (16 * _CHUNK)
            for u in range(_CHUNK):
                v = sc_v[pl.ds(base + u * 16, 16)]
                m = v > lo
                idxv = lane + (base + u * 16)
                cnt = jnp.sum(jnp.where(m, jnp.int32(1), jnp.int32(0)))
                @pl.when(off <= _CAP - 16)
                def _():
                    plsc.store_compressed(cs_v.at[pl.ds(off, 16)], v, mask=m)
                    plsc.store_compressed(ci_v.at[pl.ds(off, 16)], idxv, mask=m)
                off = jnp.minimum(off + cnt, jnp.int32(_CAP))
            return off
        ncand = lax.fori_loop(0, _NV // _CHUNK, comp_outer, jnp.int32(0))
        nv = (ncand + 15) // 16

        for j in range(_KPAD // 16):
            ss_v[pl.ds(j * 16, 16)] = jnp.full((16,), -1.0, jnp.float32)
            si_v[pl.ds(j * 16, 16)] = jnp.zeros((16,), jnp.int32)

        def pick(i, _):
            def mx(j, macc):
                return jnp.maximum(macc, cs_v[pl.ds(j * 16, 16)])
            macc = lax.fori_loop(0, nv, mx,
                                 jnp.full((16,), -2.0, jnp.float32))
            mval = jnp.max(macc)

            def mi(j, iacc):
                v = cs_v[pl.ds(j * 16, 16)]
                ix = ci_v[pl.ds(j * 16, 16)]
                return jnp.minimum(iacc, jnp.where(v == mval, ix, jnp.int32(_BIG_I)))
            imin = jnp.min(lax.fori_loop(0, nv, mi,
                                         jnp.full((16,), _BIG_I, jnp.int32)))

            plsc.store_scatter(ss_v, [jnp.full((16,), i, jnp.int32)],
                               jnp.full((16,), mval, jnp.float32), mask=mask0)
            imin_c = jnp.minimum(imin, jnp.int32(_N - 1))
            plsc.store_scatter(si_v, [jnp.full((16,), i, jnp.int32)],
                               jnp.full((16,), imin_c, jnp.int32), mask=mask0)

            def clr(j, _):
                v = cs_v[pl.ds(j * 16, 16)]
                ix = ci_v[pl.ds(j * 16, 16)]
                cs_v[pl.ds(j * 16, 16)] = jnp.where(ix == imin, -2.0, v)
                return 0
            lax.fori_loop(0, nv, clr, 0)
            return 0
        lax.fori_loop(0, _K, pick, 0)

        for j in range(_KPAD // 16):
            idxv = si_v[pl.ds(j * 16, 16)] * 2
            bs_v[pl.ds(j * 16, 16)] = plsc.load_gather(dv_v, [idxv])
            be_v[pl.ds(j * 16, 16)] = plsc.load_gather(dv_v, [idxv + 1])
            sv = ss_v[pl.ds(j * 16, 16)]
            kp_v[pl.ds(j * 16, 16)] = jnp.where(
                sv > CLS_THRESHOLD, 1.0, 0.0).astype(jnp.float32)

        def nms(i, _):
            base = (i // 16) * 16
            li = i - base
            mlane = lane == li
            bsv = bs_v[pl.ds(base, 16)]
            bev = be_v[pl.ds(base, 16)]
            kpv = kp_v[pl.ds(base, 16)]
            s_i = jnp.sum(jnp.where(mlane, bsv, 0.0))
            e_i = jnp.sum(jnp.where(mlane, bev, 0.0))
            cur = jnp.sum(jnp.where(mlane, kpv, 0.0))
            len_i = jnp.maximum(e_i - s_i, 0.0)
            @pl.when(cur > 0.5)
            def _():
                for j in range(_KPAD // 16):
                    sv = bs_v[pl.ds(j * 16, 16)]
                    ev = be_v[pl.ds(j * 16, 16)]
                    kv = kp_v[pl.ds(j * 16, 16)]
                    inter = jnp.maximum(
                        jnp.minimum(ev, e_i) - jnp.maximum(sv, s_i), 0.0)
                    union = jnp.maximum(ev - sv, 0.0) + len_i - inter
                    iou = inter / (union + 1e-9)
                    jvec = lane + (j * 16)
                    supp = jnp.logical_and(iou > OVERLAP, jvec > i)
                    kp_v[pl.ds(j * 16, 16)] = jnp.where(supp, 0.0, kv)
            return 0
        lax.fori_loop(0, _K, nms, 0)

        for j in range(_ROW // 16):
            ov_v[pl.ds(j * 16, 16)] = jnp.zeros((16,), jnp.float32)
        for j in range(_KPAD // 16):
            sl = pl.ds(j * 16, 16)
            kv = kp_v[sl]
            pos3 = (lane + j * 16) * 3
            mvalid = pos3 < (_K * 3)
            plsc.store_scatter(ov_v, [jnp.minimum(pos3, _K * 3 - 1)],
                               bs_v[sl] * kv, mask=mvalid)
            plsc.store_scatter(ov_v, [jnp.minimum(pos3 + 1, _K * 3 - 1)],
                               be_v[sl] * kv, mask=mvalid)
            plsc.store_scatter(ov_v, [jnp.minimum(pos3 + 2, _K * 3 - 1)],
                               ss_v[sl] * kv, mask=mvalid)
        pltpu.sync_copy(ov_v, out_ref.at[pl.ds((b * _CM1 + c) * _ROW, _ROW)])


def _stage_b(scores_t, decoded):
    mesh = plsc.VectorSubcoreMesh(core_axis_name="c", subcore_axis_name="s",
                                  num_cores=2, num_subcores=16)
    f = pl.kernel(
        _stage_b_body,
        out_type=jax.ShapeDtypeStruct((_B * _CM1 * _ROW,), jnp.float32),
        mesh=mesh,
        scratch_types=[
            pltpu.VMEM((_N,), jnp.float32),
            pltpu.VMEM((2 * _N,), jnp.float32),
            pltpu.VMEM((_CAPP,), jnp.float32),
            pltpu.VMEM((_CAPP,), jnp.int32),
            pltpu.VMEM((_KPAD,), jnp.float32),
            pltpu.VMEM((_KPAD,), jnp.int32),
            pltpu.VMEM((_KPAD,), jnp.float32),
            pltpu.VMEM((_KPAD,), jnp.float32),
            pltpu.VMEM((_KPAD,), jnp.float32),
            pltpu.VMEM((_ROW,), jnp.float32),
            pltpu.VMEM((16,), jnp.int32),
        ],
        compiler_params=pltpu.CompilerParams(needs_layout_passes=False),
    )
    return f(scores_t.reshape(-1), decoded.reshape(-1))


def kernel(localizations, classifications, localizations_default):
    scores_t, decoded = _stage_a(classifications, localizations,
                                 localizations_default)
    out = _stage_b(scores_t, decoded)
    out = out.reshape(_B, _CM1, _ROW)[:, :, :_K * 3]
    return out.reshape(_B, _CM1, _K, 3)


# speedup vs baseline: 4.6332x; 1.0907x over previous
import functools

import jax
import jax.numpy as jnp
from jax import lax
from jax.experimental import pallas as pl
from jax.experimental.pallas import tpu as pltpu
from jax.experimental.pallas import tpu_sc as plsc

NUM_CLASSES = 21
OVERLAP = 0.45
TOP_K = 200
CLS_THRESHOLD = 0.01

_B, _N, _C = 8, 20000, NUM_CLASSES
_NBLK = 5120


def _stage_a_body(cls_ref, loc_ref, pri_ref, sc_ref, dec_ref):
    x = cls_ref[0]
    m = jnp.max(x, axis=-1, keepdims=True)
    e = jnp.exp(x - m)
    s = jnp.sum(e, axis=-1, keepdims=True)
    p = e / s
    sc_ref[0] = jnp.transpose(p[:, 1:], (1, 0))

    l = loc_ref[0]
    pr = pri_ref[...]
    center = pr[:, 0:1] + l[:, 0:1] * 0.1 * pr[:, 1:2]
    width = pr[:, 1:2] * jnp.exp(l[:, 1:2] * 0.2)
    half = width / 2.0
    dec_ref[0] = jnp.concatenate([center - half, center + half], axis=1)


def _stage_a(cls, loc, pri):
    grid = (_B, (_N + _NBLK - 1) // _NBLK)
    return pl.pallas_call(
        _stage_a_body,
        grid=grid,
        in_specs=[
            pl.BlockSpec((1, _NBLK, _C), lambda b, n: (b, n, 0)),
            pl.BlockSpec((1, _NBLK, 2), lambda b, n: (b, n, 0)),
            pl.BlockSpec((_NBLK, 2), lambda b, n: (n, 0)),
        ],
        out_specs=[
            pl.BlockSpec((1, _C - 1, _NBLK), lambda b, n: (b, 0, n)),
            pl.BlockSpec((1, _NBLK, 2), lambda b, n: (b, n, 0)),
        ],
        out_shape=[
            jax.ShapeDtypeStruct((_B, _C - 1, _N), jnp.float32),
            jax.ShapeDtypeStruct((_B, _N, 2), jnp.float32),
        ],
    )(cls, loc, pri)


def _xla_tail(scores, decoded):
    B, N, Cm1 = scores.shape
    K = TOP_K
    scores_t = jnp.transpose(scores, (0, 2, 1))
    top_scores, top_idx = jax.lax.top_k(scores_t, K)
    decoded_b = jnp.broadcast_to(decoded[:, None, :, :], (B, Cm1, N, 2))
    boxes = jnp.take_along_axis(decoded_b, top_idx[..., None], axis=2)
    valid = top_scores > CLS_THRESHOLD
    s = boxes[..., 0]
    e = boxes[..., 1]
    lengths = jnp.clip(e - s, 0.0)
    inter = jnp.clip(
        jnp.minimum(e[..., :, None], e[..., None, :])
        - jnp.maximum(s[..., :, None], s[..., None, :]), 0.0)
    union = lengths[..., :, None] + lengths[..., None, :] - inter
    iou = inter / (union + 1e-9)
    keep = valid
    idxr = jnp.arange(K)
    for i in range(K):
        cur = keep[..., i]
        supp = (iou[..., i, :] > OVERLAP) & (idxr > i)
        keep = keep & ~(cur[..., None] & supp)
    out = jnp.concatenate([boxes, top_scores[..., None]], axis=-1)
    return out * keep[..., None].astype(out.dtype)


_CM1 = _C - 1
_K = TOP_K
_KPAD = 208
_CAP = 512
_CAPP = _CAP + 16
_ROW = 640
_NV = _N // 16
_CHUNK = 10
_BIG_I = 2**30


def _count_gt(sc_ref, thr):
    def outer(i, acc):
        base = i * (16 * _CHUNK)
        for u in range(_CHUNK):
            v = sc_ref[pl.ds(base + u * 16, 16)]
            acc = acc + jnp.where(v > thr, jnp.int32(1), jnp.int32(0))
        return acc
    acc = lax.fori_loop(0, _NV // _CHUNK, outer,
                        jnp.zeros((16,), jnp.int32))
    return jnp.sum(acc)


def _stage_b_body(sct_ref, dec_ref, out_ref,
                  sc_v, dv_v, cs_v, ci_v, ss_v, si_v,
                  bs_v, be_v, kp_v, ov_v, tb_v):
    nc = 2
    wid = lax.axis_index("s") * nc + lax.axis_index("c")
    b = wid // 4
    cg = wid % 4

    lane = lax.iota(jnp.int32, 16)
    mask0 = lane == 0

    pltpu.sync_copy(dec_ref.at[pl.ds(b * 2 * _N, 2 * _N)], dv_v)

    for k in range(_CM1 // 4):
        c = cg * 5 + k
        pltpu.sync_copy(sct_ref.at[pl.ds((b * _CM1 + c) * _N, _N)], sc_v)

        def bs_body(_, st):
            lo, hi, cnt_lo = st
            done = cnt_lo <= 448
            mid = (lo + hi) * 0.5
            @pl.when(jnp.logical_not(done))
            def _():
                tb_v[pl.ds(0, 16)] = jnp.full(
                    (16,), _count_gt(sc_v, mid), jnp.int32)
            cnt = tb_v[pl.ds(0, 16)][0]
            ge = cnt >= 200
            lo2 = jnp.where(jnp.logical_or(done, jnp.logical_not(ge)), lo, mid)
            hi2 = jnp.where(jnp.logical_or(done, ge), hi, mid)
            cnt2 = jnp.where(jnp.logical_or(done, jnp.logical_not(ge)),
                             cnt_lo, cnt)
            return (lo2, hi2, cnt2)

        lo, _, _ = lax.fori_loop(
            0, 26, bs_body,
            (jnp.float32(0.0), jnp.float32(1.0), jnp.int32(_N)))

        for j in range(_CAPP // 16):
            cs_v[pl.ds(j * 16, 16)] = jnp.full((16,), -1.0, jnp.float32)
            ci_v[pl.ds(j * 16, 16)] = jnp.full((16,), _BIG_I, jnp.int32)

        def comp_outer(i, off):
            base = i * (16 * _CHUNK)
            for u in range(_CHUNK):
                v = sc_v[pl.ds(base + u * 16, 16)]
                m = v > lo
                idxv = lane + (base + u * 16)
                cnt = jnp.sum(jnp.where(m, jnp.int32(1), jnp.int32(0)))
                @pl.when(off <= _CAP - 16)
                def _():
                    plsc.store_compressed(cs_v.at[pl.ds(off, 16)], v, mask=m)
                    plsc.store_compressed(ci_v.at[pl.ds(off, 16)], idxv, mask=m)
                off = jnp.minimum(off + cnt, jnp.int32(_CAP))
            return off
        ncand = lax.fori_loop(0, _NV // _CHUNK, comp_outer, jnp.int32(0))
        nv = (ncand + 15) // 16

        for j in range(_KPAD // 16):
            ss_v[pl.ds(j * 16, 16)] = jnp.full((16,), -1.0, jnp.float32)
            si_v[pl.ds(j * 16, 16)] = jnp.zeros((16,), jnp.int32)

        def pick(i, _):
            def mx(j, st):
                macc, iacc = st
                v = cs_v[pl.ds(j * 16, 16)]
                ix = ci_v[pl.ds(j * 16, 16)]
                gt = v > macc
                eq = v == macc
                macc2 = jnp.where(gt, v, macc)
                iacc2 = jnp.where(gt, ix,
                                  jnp.where(eq, jnp.minimum(iacc, ix), iacc))
                return (macc2, iacc2)
            macc, iacc = lax.fori_loop(
                0, nv, mx,
                (jnp.full((16,), -2.0, jnp.float32),
                 jnp.full((16,), _BIG_I, jnp.int32)))
            mval = jnp.max(macc)
            imin = jnp.min(jnp.where(macc == mval, iacc, jnp.int32(_BIG_I)))

            plsc.store_scatter(ss_v, [jnp.full((16,), i, jnp.int32)],
                               jnp.full((16,), mval, jnp.float32), mask=mask0)
            imin_c = jnp.minimum(imin, jnp.int32(_N - 1))
            plsc.store_scatter(si_v, [jnp.full((16,), i, jnp.int32)],
                               jnp.full((16,), imin_c, jnp.int32), mask=mask0)

            def clr(j, _):
                v = cs_v[pl.ds(j * 16, 16)]
                ix = ci_v[pl.ds(j * 16, 16)]
                cs_v[pl.ds(j * 16, 16)] = jnp.where(ix == imin, -2.0, v)
                return 0
            lax.fori_loop(0, nv, clr, 0)
            return 0
        lax.fori_loop(0, _K, pick, 0)

        for j in range(_KPAD // 16):
            idxv = si_v[pl.ds(j * 16, 16)] * 2
            bs_v[pl.ds(j * 16, 16)] = plsc.load_gather(dv_v, [idxv])
            be_v[pl.ds(j * 16, 16)] = plsc.load_gather(dv_v, [idxv + 1])
            sv = ss_v[pl.ds(j * 16, 16)]
            kp_v[pl.ds(j * 16, 16)] = jnp.where(
                sv > CLS_THRESHOLD, 1.0, 0.0).astype(jnp.float32)

        def nms(i, _):
            base = (i // 16) * 16
            li = i - base
            mlane = lane == li
            bsv = bs_v[pl.ds(base, 16)]
            bev = be_v[pl.ds(base, 16)]
            kpv = kp_v[pl.ds(base, 16)]
            s_i = jnp.sum(jnp.where(mlane, bsv, 0.0))
            e_i = jnp.sum(jnp.where(mlane, bev, 0.0))
            cur = jnp.sum(jnp.where(mlane, kpv, 0.0))
            len_i = jnp.maximum(e_i - s_i, 0.0)
            @pl.when(cur > 0.5)
            def _():
                for j in range(_KPAD // 16):
                    sv = bs_v[pl.ds(j * 16, 16)]
                    ev = be_v[pl.ds(j * 16, 16)]
                    kv = kp_v[pl.ds(j * 16, 16)]
                    inter = jnp.maximum(
                        jnp.minimum(ev, e_i) - jnp.maximum(sv, s_i), 0.0)
                    union = jnp.maximum(ev - sv, 0.0) + len_i - inter
                    iou = inter / (union + 1e-9)
                    jvec = lane + (j * 16)
                    supp = jnp.logical_and(iou > OVERLAP, jvec > i)
                    kp_v[pl.ds(j * 16, 16)] = jnp.where(supp, 0.0, kv)
            return 0
        lax.fori_loop(0, _K, nms, 0)

        for j in range(_ROW // 16):
            ov_v[pl.ds(j * 16, 16)] = jnp.zeros((16,), jnp.float32)
        for j in range(_KPAD // 16):
            sl = pl.ds(j * 16, 16)
            kv = kp_v[sl]
            pos3 = (lane + j * 16) * 3
            mvalid = pos3 < (_K * 3)
            plsc.store_scatter(ov_v, [jnp.minimum(pos3, _K * 3 - 1)],
                               bs_v[sl] * kv, mask=mvalid)
            plsc.store_scatter(ov_v, [jnp.minimum(pos3 + 1, _K * 3 - 1)],
                               be_v[sl] * kv, mask=mvalid)
            plsc.store_scatter(ov_v, [jnp.minimum(pos3 + 2, _K * 3 - 1)],
                               ss_v[sl] * kv, mask=mvalid)
        pltpu.sync_copy(ov_v, out_ref.at[pl.ds((b * _CM1 + c) * _ROW, _ROW)])


def _stage_b(scores_t, decoded):
    mesh = plsc.VectorSubcoreMesh(core_axis_name="c", subcore_axis_name="s",
                                  num_cores=2, num_subcores=16)
    f = pl.kernel(
        _stage_b_body,
        out_type=jax.ShapeDtypeStruct((_B * _CM1 * _ROW,), jnp.float32),
        mesh=mesh,
        scratch_types=[
            pltpu.VMEM((_N,), jnp.float32),
            pltpu.VMEM((2 * _N,), jnp.float32),
            pltpu.VMEM((_CAPP,), jnp.float32),
            pltpu.VMEM((_CAPP,), jnp.int32),
            pltpu.VMEM((_KPAD,), jnp.float32),
            pltpu.VMEM((_KPAD,), jnp.int32),
            pltpu.VMEM((_KPAD,), jnp.float32),
            pltpu.VMEM((_KPAD,), jnp.float32),
            pltpu.VMEM((_KPAD,), jnp.float32),
            pltpu.VMEM((_ROW,), jnp.float32),
            pltpu.VMEM((16,), jnp.int32),
        ],
        compiler_params=pltpu.CompilerParams(needs_layout_passes=False),
    )
    return f(scores_t.reshape(-1), decoded.reshape(-1))


def kernel(localizations, classifications, localizations_default):
    scores_t, decoded = _stage_a(classifications, localizations,
                                 localizations_default)
    out = _stage_b(scores_t, decoded)
    out = out.reshape(_B, _CM1, _ROW)[:, :, :_K * 3]
    return out.reshape(_B, _CM1, _K, 3)


# speedup vs baseline: 4.6335x; 1.0000x over previous
import jax
import jax.numpy as jnp
from jax import lax
from jax.experimental import pallas as pl
from jax.experimental.pallas import tpu as pltpu
from jax.experimental.pallas import tpu_sc as plsc

NUM_CLASSES = 21
OVERLAP = 0.45
TOP_K = 200
CLS_THRESHOLD = 0.01

_B, _N, _C = 8, 20000, NUM_CLASSES
_NBLK = 5120


def _stage_a_body(cls_ref, loc_ref, pri_ref, sc_ref, dec_ref):
    x = cls_ref[0]
    m = jnp.max(x, axis=-1, keepdims=True)
    e = jnp.exp(x - m)
    s = jnp.sum(e, axis=-1, keepdims=True)
    p = e / s
    sc_ref[0] = jnp.transpose(p[:, 1:], (1, 0))

    l = loc_ref[0]
    pr = pri_ref[...]
    center = pr[:, 0:1] + l[:, 0:1] * 0.1 * pr[:, 1:2]
    width = pr[:, 1:2] * jnp.exp(l[:, 1:2] * 0.2)
    half = width / 2.0
    dec_ref[0] = jnp.concatenate([center - half, center + half], axis=1)


def _stage_a(cls, loc, pri):
    grid = (_B, (_N + _NBLK - 1) // _NBLK)
    return pl.pallas_call(
        _stage_a_body,
        grid=grid,
        in_specs=[
            pl.BlockSpec((1, _NBLK, _C), lambda b, n: (b, n, 0)),
            pl.BlockSpec((1, _NBLK, 2), lambda b, n: (b, n, 0)),
            pl.BlockSpec((_NBLK, 2), lambda b, n: (n, 0)),
        ],
        out_specs=[
            pl.BlockSpec((1, _C - 1, _NBLK), lambda b, n: (b, 0, n)),
            pl.BlockSpec((1, _NBLK, 2), lambda b, n: (b, n, 0)),
        ],
        out_shape=[
            jax.ShapeDtypeStruct((_B, _C - 1, _N), jnp.float32),
            jax.ShapeDtypeStruct((_B, _N, 2), jnp.float32),
        ],
    )(cls, loc, pri)


_CM1 = _C - 1
_K = TOP_K
_KPAD = 208
_CAP = 512
_CAPP = _CAP + 16
_ROW = 640
_NV = _N // 16
_CHUNK = 10
_BIG_I = 2**30


def _count_gt(sc_ref, thr):
    def outer(i, acc):
        base = i * (16 * _CHUNK)
        for u in range(_CHUNK):
            v = sc_ref[pl.ds(base + u * 16, 16)]
            acc = acc + jnp.where(v > thr, jnp.int32(1), jnp.int32(0))
        return acc
    acc = lax.fori_loop(0, _NV // _CHUNK, outer,
                        jnp.zeros((16,), jnp.int32))
    return jnp.sum(acc)


def _stage_b_body(sct_ref, dec_ref, out_ref,
                  sc_v, dv_v, cs_v, ci_v, ss_v, si_v,
                  bs_v, be_v, kp_v, ov_v, tb_v):
    nc = 2
    wid = lax.axis_index("s") * nc + lax.axis_index("c")
    b = wid // 4
    cg = wid % 4

    lane = lax.iota(jnp.int32, 16)
    mask0 = lane == 0

    pltpu.sync_copy(dec_ref.at[pl.ds(b * 2 * _N, 2 * _N)], dv_v)

    for k in range(_CM1 // 4):
        c = cg * 5 + k
        pltpu.sync_copy(sct_ref.at[pl.ds((b * _CM1 + c) * _N, _N)], sc_v)

        def bs_body(_, st):
            lo, hi, cnt_lo = st
            done = cnt_lo <= 448
            mid = (lo + hi) * 0.5
            @pl.when(jnp.logical_not(done))
            def _():
                tb_v[pl.ds(0, 16)] = jnp.full(
                    (16,), _count_gt(sc_v, mid), jnp.int32)
            cnt = tb_v[pl.ds(0, 16)][0]
            ge = cnt >= 200
            lo2 = jnp.where(jnp.logical_or(done, jnp.logical_not(ge)), lo, mid)
            hi2 = jnp.where(jnp.logical_or(done, ge), hi, mid)
            cnt2 = jnp.where(jnp.logical_or(done, jnp.logical_not(ge)),
                             cnt_lo, cnt)
            return (lo2, hi2, cnt2)

        lo, _, _ = lax.fori_loop(
            0, 26, bs_body,
            (jnp.float32(0.0), jnp.float32(1.0), jnp.int32(_N)))

        for j in range(_CAPP // 16):
            cs_v[pl.ds(j * 16, 16)] = jnp.full((16,), -1.0, jnp.float32)
            ci_v[pl.ds(j * 16, 16)] = jnp.full((16,), _BIG_I, jnp.int32)

        def comp_outer(i, off):
            base = i * (16 * _CHUNK)
            for u in range(_CHUNK):
                v = sc_v[pl.ds(base + u * 16, 16)]
                m = v > lo
                idxv = lane + (base + u * 16)
                cnt = jnp.sum(jnp.where(m, jnp.int32(1), jnp.int32(0)))
                @pl.when(off <= _CAP - 16)
                def _():
                    plsc.store_compressed(cs_v.at[pl.ds(off, 16)], v, mask=m)
                    plsc.store_compressed(ci_v.at[pl.ds(off, 16)], idxv, mask=m)
                off = jnp.minimum(off + cnt, jnp.int32(_CAP))
            return off
        ncand = lax.fori_loop(0, _NV // _CHUNK, comp_outer, jnp.int32(0))
        nv = (ncand + 15) // 16

        for j in range(_KPAD // 16):
            ss_v[pl.ds(j * 16, 16)] = jnp.full((16,), -1.0, jnp.float32)
            si_v[pl.ds(j * 16, 16)] = jnp.zeros((16,), jnp.int32)

        def pick(i, _):
            def mx(j, st):
                macc, iacc = st
                v = cs_v[pl.ds(j * 16, 16)]
                ix = ci_v[pl.ds(j * 16, 16)]
                gt = v > macc
                eq = v == macc
                macc2 = jnp.where(gt, v, macc)
                iacc2 = jnp.where(gt, ix,
                                  jnp.where(eq, jnp.minimum(iacc, ix), iacc))
                return (macc2, iacc2)
            macc, iacc = lax.fori_loop(
                0, nv, mx,
                (jnp.full((16,), -2.0, jnp.float32),
                 jnp.full((16,), _BIG_I, jnp.int32)))
            mval = jnp.max(macc)
            imin = jnp.min(jnp.where(macc == mval, iacc, jnp.int32(_BIG_I)))

            plsc.store_scatter(ss_v, [jnp.full((16,), i, jnp.int32)],
                               jnp.full((16,), mval, jnp.float32), mask=mask0)
            imin_c = jnp.minimum(imin, jnp.int32(_N - 1))
            plsc.store_scatter(si_v, [jnp.full((16,), i, jnp.int32)],
                               jnp.full((16,), imin_c, jnp.int32), mask=mask0)

            def clr(j, _):
                v = cs_v[pl.ds(j * 16, 16)]
                ix = ci_v[pl.ds(j * 16, 16)]
                cs_v[pl.ds(j * 16, 16)] = jnp.where(ix == imin, -2.0, v)
                return 0
            lax.fori_loop(0, nv, clr, 0)
            return 0
        lax.fori_loop(0, _K, pick, 0)

        for j in range(_KPAD // 16):
            idxv = si_v[pl.ds(j * 16, 16)] * 2
            bs_v[pl.ds(j * 16, 16)] = plsc.load_gather(dv_v, [idxv])
            be_v[pl.ds(j * 16, 16)] = plsc.load_gather(dv_v, [idxv + 1])
            sv = ss_v[pl.ds(j * 16, 16)]
            kp_v[pl.ds(j * 16, 16)] = jnp.where(
                sv > CLS_THRESHOLD, 1.0, 0.0).astype(jnp.float32)

        def nms(i, _):
            base = (i // 16) * 16
            li = i - base
            mlane = lane == li
            bsv = bs_v[pl.ds(base, 16)]
            bev = be_v[pl.ds(base, 16)]
            kpv = kp_v[pl.ds(base, 16)]
            s_i = jnp.sum(jnp.where(mlane, bsv, 0.0))
            e_i = jnp.sum(jnp.where(mlane, bev, 0.0))
            cur = jnp.sum(jnp.where(mlane, kpv, 0.0))
            len_i = jnp.maximum(e_i - s_i, 0.0)
            @pl.when(cur > 0.5)
            def _():
                for j in range(_KPAD // 16):
                    sv = bs_v[pl.ds(j * 16, 16)]
                    ev = be_v[pl.ds(j * 16, 16)]
                    kv = kp_v[pl.ds(j * 16, 16)]
                    inter = jnp.maximum(
                        jnp.minimum(ev, e_i) - jnp.maximum(sv, s_i), 0.0)
                    union = jnp.maximum(ev - sv, 0.0) + len_i - inter
                    iou = inter / (union + 1e-9)
                    jvec = lane + (j * 16)
                    supp = jnp.logical_and(iou > OVERLAP, jvec > i)
                    kp_v[pl.ds(j * 16, 16)] = jnp.where(supp, 0.0, kv)
            return 0
        lax.fori_loop(0, _K, nms, 0)

        for j in range(_ROW // 16):
            ov_v[pl.ds(j * 16, 16)] = jnp.zeros((16,), jnp.float32)
        for j in range(_KPAD // 16):
            sl = pl.ds(j * 16, 16)
            kv = kp_v[sl]
            pos3 = (lane + j * 16) * 3
            mvalid = pos3 < (_K * 3)
            plsc.store_scatter(ov_v, [jnp.minimum(pos3, _K * 3 - 1)],
                               bs_v[sl] * kv, mask=mvalid)
            plsc.store_scatter(ov_v, [jnp.minimum(pos3 + 1, _K * 3 - 1)],
                               be_v[sl] * kv, mask=mvalid)
            plsc.store_scatter(ov_v, [jnp.minimum(pos3 + 2, _K * 3 - 1)],
                               ss_v[sl] * kv, mask=mvalid)
        pltpu.sync_copy(ov_v, out_ref.at[pl.ds((b * _CM1 + c) * _ROW, _ROW)])


def _stage_b(scores_t, decoded):
    mesh = plsc.VectorSubcoreMesh(core_axis_name="c", subcore_axis_name="s",
                                  num_cores=2, num_subcores=16)
    f = pl.kernel(
        _stage_b_body,
        out_type=jax.ShapeDtypeStruct((_B * _CM1 * _ROW,), jnp.float32),
        mesh=mesh,
        scratch_types=[
            pltpu.VMEM((_N,), jnp.float32),
            pltpu.VMEM((2 * _N,), jnp.float32),
            pltpu.VMEM((_CAPP,), jnp.float32),
            pltpu.VMEM((_CAPP,), jnp.int32),
            pltpu.VMEM((_KPAD,), jnp.float32),
            pltpu.VMEM((_KPAD,), jnp.int32),
            pltpu.VMEM((_KPAD,), jnp.float32),
            pltpu.VMEM((_KPAD,), jnp.float32),
            pltpu.VMEM((_KPAD,), jnp.float32),
            pltpu.VMEM((_ROW,), jnp.float32),
            pltpu.VMEM((16,), jnp.int32),
        ],
        compiler_params=pltpu.CompilerParams(needs_layout_passes=False),
    )
    return f(scores_t.reshape(-1), decoded.reshape(-1))


def kernel(localizations, classifications, localizations_default):
    scores_t, decoded = _stage_a(classifications, localizations,
                                 localizations_default)
    out = _stage_b(scores_t, decoded)
    out = out.reshape(_B, _CM1, _ROW)[:, :, :_K * 3]
    return out.reshape(_B, _CM1, _K, 3)
